# baseline (speedup 1.0000x reference)
import functools

import jax
import jax.numpy as jnp
from jax import lax
from jax.experimental import pallas as pl
from jax.experimental.pallas import tpu as pltpu
from jax.experimental.pallas import tpu_sc as plsc

_SHAPE = (4096, 4096)
_FLAT = _SHAPE[0] * _SHAPE[1]
_K = 1048576
_NC, _NS = 2, 16
_NW = _NC * _NS
_BLK = 32768
_CH = _BLK * _NS
_NCHUNK = _FLAT // _NC // _CH
_NBLK = _FLAT // _BLK
_NR = 48
_PROWS = _K // 128
_PPAD = _PROWS + _NR

_mesh = plsc.VectorSubcoreMesh(core_axis_name="c", subcore_axis_name="s")


@functools.partial(
    pl.kernel,
    out_type=jax.ShapeDtypeStruct((_FLAT,), jnp.float32),
    mesh=_mesh,
    compiler_params=pltpu.CompilerParams(needs_layout_passes=False),
    scratch_types=[
        pltpu.MemorySpace.VMEM_SHARED((_CH + 16,), jnp.float32),
        pltpu.MemorySpace.VMEM_SHARED((_CH + 16,), jnp.float32),
        pltpu.VMEM((_NR, 128), jnp.int32),
        pltpu.VMEM((_NR, 128), jnp.int32),
        pltpu.VMEM((_NR, 128), jnp.float32),
        pltpu.VMEM((_NR, 128), jnp.float32),
        pltpu.VMEM((_NR, 128), jnp.int32),
        pltpu.VMEM((32,), jnp.int32),
        pltpu.SemaphoreType.DMA,
        pltpu.SemaphoreType.DMA,
        pltpu.SemaphoreType.DMA,
        pltpu.SemaphoreType.DMA,
        pltpu.SemaphoreType.DMA,
        pltpu.SemaphoreType.DMA,
    ],
)
def _sc_scatter_add(tensor_hbm, val2d_hbm, idx2d_hbm, table_hbm, out_hbm,
                    sp_a, sp_b, idx_a, idx_b, val_a, val_b, lidx_v, bnd_v,
                    dld_a, dld_b, dst_a, dst_b, pld_a, pld_b):
    c = lax.axis_index("c")
    s = lax.axis_index("s")
    row_id = s * _NC + c
    sl_lo = s * _BLK

    pltpu.sync_copy(table_hbm.at[pl.ds(row_id * 32, 32)], bnd_v)
    bv0 = bnd_v[pl.ds(0, 16)]
    bv1 = bnd_v[pl.ds(16, 16)]

    trash = lax.iota(jnp.int32, 16) + _CH
    blk_u = jnp.uint32(_BLK)

    sps = (sp_a, sp_b)
    idxs = (idx_a, idx_b)
    vals = (val_a, val_b)
    dld = (dld_a, dld_b)
    dst = (dst_a, dst_b)
    pld = (pld_a, pld_b)

    dld_desc = [None, None]
    dst_desc = [None, None]
    pld_desc = [None, None]

    def start_dense_load(k):
        cur = k % 2
        hbm_lo = (c * _NCHUNK + k) * _CH + sl_lo
        dld_desc[cur] = pltpu.async_copy(
            tensor_hbm.at[pl.ds(hbm_lo, _BLK)],
            sps[cur].at[pl.ds(sl_lo, _BLK)], dld[cur])

    def start_pair_load(k, r0):
        cur = k % 2
        d1 = pltpu.async_copy(idx2d_hbm.at[pl.ds(r0, _NR)], idxs[cur], pld[cur])
        d2 = pltpu.async_copy(val2d_hbm.at[pl.ds(r0, _NR)], vals[cur], pld[cur])
        pld_desc[cur] = (d1, d2)

    def scatter_batch(nrows, idxb, valb, spb, blk_lo):
        def body(row, carry):
            for g in range(8):
                iv = idxb[row, pl.ds(g * 16, 16)]
                d = iv - blk_lo
                inb = plsc.bitcast(d, jnp.uint32) < blk_u
                e = d + sl_lo
                lv = jnp.where(inb, e, trash)
                lidx_v[row, pl.ds(g * 16, 16)] = lv
            pltpu.sync_copy(valb.at[row], spb.at[lidx_v.at[row]], add=True)
            return carry
        lax.fori_loop(0, nrows, body, 0)

    r0s = [pl.multiple_of((bv0[k] >> 10) << 3, 8) for k in range(_NCHUNK)]
    start_dense_load(0)
    start_pair_load(0, r0s[0])

    for k in range(_NCHUNK):
        cur = k % 2
        nxt = 1 - cur
        if k + 1 < _NCHUNK:
            if dst_desc[nxt] is not None:
                dst_desc[nxt].wait()
                dst_desc[nxt] = None
            start_dense_load(k + 1)
            start_pair_load(k + 1, r0s[k + 1])

        chunk_lo = (c * _NCHUNK + k) * _CH
        blk_lo = chunk_lo + sl_lo
        p1 = bv1[k]
        r0 = r0s[k]
        nr = ((p1 + 127) >> 7) - r0

        dld_desc[cur].wait()
        for d in pld_desc[cur]:
            d.wait()
        pld_desc[cur] = None

        scatter_batch(jnp.minimum(nr, _NR), idxs[cur], vals[cur], sps[cur],
                      blk_lo)

        nbatch = (nr + _NR - 1) // _NR

        def rem_body(j, carry, cur=cur, r0=r0, nr=nr, blk_lo=blk_lo):
            rb = pl.multiple_of(r0 + j * _NR, 8)
            pltpu.sync_copy(idx2d_hbm.at[pl.ds(rb, _NR)], idxs[cur])
            pltpu.sync_copy(val2d_hbm.at[pl.ds(rb, _NR)], vals[cur])
            scatter_batch(jnp.minimum(nr - j * _NR, _NR), idxs[cur],
                          vals[cur], sps[cur], blk_lo)
            return carry

        lax.fori_loop(1, nbatch, rem_body, 0)

        dst_desc[cur] = pltpu.async_copy(
            sps[cur].at[pl.ds(sl_lo, _BLK)],
            out_hbm.at[pl.ds((c * _NCHUNK + k) * _CH + sl_lo, _BLK)],
            dst[cur])

    for d in dst_desc:
        if d is not None:
            d.wait()


def kernel(tensor, values, indices):
    flat = tensor.reshape(-1)
    queries = jnp.arange(_NBLK + 1, dtype=jnp.int32) * _BLK
    bounds = jnp.searchsorted(indices, queries, side="left").astype(jnp.int32)
    w = jnp.arange(_NW)
    s_ = w // _NC
    c_ = w % _NC
    k_ = jnp.arange(_NCHUNK)
    ids = c_[:, None] * (_NBLK // _NC) + k_[None, :] * _NS + s_[:, None]
    table = jnp.concatenate([bounds[ids], bounds[ids + 1]],
                            axis=1).reshape(-1)

    pad_i = jnp.full((_NR * 128,), _FLAT, jnp.int32)
    pad_v = jnp.zeros((_NR * 128,), jnp.float32)
    idx2d = jnp.concatenate([indices, pad_i]).reshape(_PPAD, 128)
    val2d = jnp.concatenate([values, pad_v]).reshape(_PPAD, 128)

    out = _sc_scatter_add(flat, val2d, idx2d, table)
    return out.reshape(_SHAPE)


# speedup vs baseline: 1.0573x; 1.0573x over previous
import functools

import jax
import jax.numpy as jnp
from jax import lax
from jax.experimental import pallas as pl
from jax.experimental.pallas import tpu as pltpu
from jax.experimental.pallas import tpu_sc as plsc

_SHAPE = (4096, 4096)
_FLAT = _SHAPE[0] * _SHAPE[1]
_K = 1048576
_NC, _NS = 2, 16
_NW = _NC * _NS
_BLK = 32768
_CH = _BLK * _NS
_NCHUNK = _FLAT // _NC // _CH
_NBLK = _FLAT // _BLK
_NR = 48
_PROWS = _K // 128
_PPAD = _PROWS + _NR

_mesh = plsc.VectorSubcoreMesh(core_axis_name="c", subcore_axis_name="s")


@functools.partial(
    pl.kernel,
    out_type=jax.ShapeDtypeStruct((_FLAT,), jnp.float32),
    mesh=_mesh,
    compiler_params=pltpu.CompilerParams(needs_layout_passes=False),
    scratch_types=[
        pltpu.MemorySpace.VMEM_SHARED((_CH + 16,), jnp.float32),
        pltpu.MemorySpace.VMEM_SHARED((_CH + 16,), jnp.float32),
        pltpu.VMEM((_NR, 128), jnp.int32),
        pltpu.VMEM((_NR, 128), jnp.int32),
        pltpu.VMEM((_NR, 128), jnp.float32),
        pltpu.VMEM((_NR, 128), jnp.float32),
        pltpu.VMEM((_NR, 128), jnp.int32),
        pltpu.VMEM((32,), jnp.int32),
        pltpu.SemaphoreType.DMA,
        pltpu.SemaphoreType.DMA,
        pltpu.SemaphoreType.DMA,
        pltpu.SemaphoreType.DMA,
        pltpu.SemaphoreType.DMA,
        pltpu.SemaphoreType.DMA,
        pltpu.SemaphoreType.DMA,
    ],
)
def _sc_scatter_add(tensor_hbm, val2d_hbm, idx2d_hbm, table_hbm, out_hbm,
                    sp_a, sp_b, idx_a, idx_b, val_a, val_b, lidx_v, bnd_v,
                    dld_a, dld_b, dst_a, dst_b, pld_a, pld_b, ssem):
    c = lax.axis_index("c")
    s = lax.axis_index("s")
    row_id = s * _NC + c
    sl_lo = s * _BLK

    pltpu.sync_copy(table_hbm.at[pl.ds(row_id * 32, 32)], bnd_v)
    bv0 = bnd_v[pl.ds(0, 16)]
    bv1 = bnd_v[pl.ds(16, 16)]

    trash = lax.iota(jnp.int32, 16) + _CH
    blk_u = jnp.uint32(_BLK)

    sps = (sp_a, sp_b)
    idxs = (idx_a, idx_b)
    vals = (val_a, val_b)
    dld = (dld_a, dld_b)
    dst = (dst_a, dst_b)
    pld = (pld_a, pld_b)

    dld_desc = [None, None]
    dst_desc = [None, None]
    pld_desc = [None, None]

    def start_dense_load(k):
        cur = k % 2
        hbm_lo = (c * _NCHUNK + k) * _CH + sl_lo
        dld_desc[cur] = pltpu.async_copy(
            tensor_hbm.at[pl.ds(hbm_lo, _BLK)],
            sps[cur].at[pl.ds(sl_lo, _BLK)], dld[cur])

    def start_pair_load(k, r0):
        cur = k % 2
        d1 = pltpu.async_copy(idx2d_hbm.at[pl.ds(r0, _NR)], idxs[cur], pld[cur])
        d2 = pltpu.async_copy(val2d_hbm.at[pl.ds(r0, _NR)], vals[cur], pld[cur])
        pld_desc[cur] = (d1, d2)

    def scatter_batch(nrows, idxb, valb, spb, blk_lo):
        def body(row, carry):
            for g in range(8):
                iv = idxb[row, pl.ds(g * 16, 16)]
                d = iv - blk_lo
                inb = plsc.bitcast(d, jnp.uint32) < blk_u
                e = d + sl_lo
                lv = jnp.where(inb, e, trash)
                lidx_v[row, pl.ds(g * 16, 16)] = lv

            @pl.when(row >= 1)
            def _wait_prev():
                pltpu.make_async_copy(
                    valb.at[row - 1], spb.at[lidx_v.at[row - 1]], ssem).wait()

            pltpu.async_copy(valb.at[row], spb.at[lidx_v.at[row]], ssem,
                             add=True)
            return carry
        lax.fori_loop(0, nrows, body, 0)

        @pl.when(nrows >= 1)
        def _wait_last():
            pltpu.make_async_copy(
                valb.at[nrows - 1], spb.at[lidx_v.at[nrows - 1]], ssem).wait()

    r0s = [pl.multiple_of((bv0[k] >> 10) << 3, 8) for k in range(_NCHUNK)]
    start_dense_load(0)
    start_pair_load(0, r0s[0])

    for k in range(_NCHUNK):
        cur = k % 2
        nxt = 1 - cur
        if k + 1 < _NCHUNK:
            if dst_desc[nxt] is not None:
                dst_desc[nxt].wait()
                dst_desc[nxt] = None
            start_dense_load(k + 1)
            start_pair_load(k + 1, r0s[k + 1])

        chunk_lo = (c * _NCHUNK + k) * _CH
        blk_lo = chunk_lo + sl_lo
        p1 = bv1[k]
        r0 = r0s[k]
        nr = ((p1 + 127) >> 7) - r0

        dld_desc[cur].wait()
        for d in pld_desc[cur]:
            d.wait()
        pld_desc[cur] = None

        scatter_batch(jnp.minimum(nr, _NR), idxs[cur], vals[cur], sps[cur],
                      blk_lo)

        nbatch = (nr + _NR - 1) // _NR

        def rem_body(j, carry, cur=cur, r0=r0, nr=nr, blk_lo=blk_lo):
            rb = pl.multiple_of(r0 + j * _NR, 8)
            pltpu.sync_copy(idx2d_hbm.at[pl.ds(rb, _NR)], idxs[cur])
            pltpu.sync_copy(val2d_hbm.at[pl.ds(rb, _NR)], vals[cur])
            scatter_batch(jnp.minimum(nr - j * _NR, _NR), idxs[cur],
                          vals[cur], sps[cur], blk_lo)
            return carry

        lax.fori_loop(1, nbatch, rem_body, 0)

        dst_desc[cur] = pltpu.async_copy(
            sps[cur].at[pl.ds(sl_lo, _BLK)],
            out_hbm.at[pl.ds((c * _NCHUNK + k) * _CH + sl_lo, _BLK)],
            dst[cur])

    for d in dst_desc:
        if d is not None:
            d.wait()


def kernel(tensor, values, indices):
    flat = tensor.reshape(-1)
    queries = jnp.arange(_NBLK + 1, dtype=jnp.int32) * _BLK
    bounds = jnp.searchsorted(indices, queries, side="left").astype(jnp.int32)
    w = jnp.arange(_NW)
    s_ = w // _NC
    c_ = w % _NC
    k_ = jnp.arange(_NCHUNK)
    ids = c_[:, None] * (_NBLK // _NC) + k_[None, :] * _NS + s_[:, None]
    table = jnp.concatenate([bounds[ids], bounds[ids + 1]],
                            axis=1).reshape(-1)

    pad_i = jnp.full((_NR * 128,), _FLAT, jnp.int32)
    pad_v = jnp.zeros((_NR * 128,), jnp.float32)
    idx2d = jnp.concatenate([indices, pad_i]).reshape(_PPAD, 128)
    val2d = jnp.concatenate([values, pad_v]).reshape(_PPAD, 128)

    out = _sc_scatter_add(flat, val2d, idx2d, table)
    return out.reshape(_SHAPE)
